# Initial kernel scaffold; baseline (speedup 1.0000x reference)
import jax
import jax.numpy as jnp
from jax.experimental import pallas as pl


def kernel(x, edge_index, W1_rel, W1_root, b1, W2_rel, W2_root, b2, W3, a_src, a_dst, b3, Wres, bres, Wfc, bfc):
    raise NotImplementedError("write your pallas kernel here")


# speedup vs baseline: 72.5726x; 72.5726x over previous
import functools

import jax
import jax.numpy as jnp
from jax import lax
from jax.experimental import pallas as pl
from jax.experimental.pallas import tpu as pltpu
from jax.experimental.pallas import tpu_sc as plsc

N = 256
E = 65280
IN = 7
HID = 128
HEADS = 2
OUT = N * 6

NC, NS = 2, 16
NW = NC * NS
EPAD = 65536
CHUNK = EPAD // NW
VECS = CHUNK // 16
ROWS = 264
ACC = ROWS * N
ZSL = ACC // NS

def _edge_counts_body(src_hbm, dst_hbm, out_hbm, src_v, dst_v, idx_v, ones_v,
                      zero_v, acc_sh):
    cid = lax.axis_index("c")
    sid = lax.axis_index("s")
    wid = sid * NC + cid
    base = wid * CHUNK

    pltpu.sync_copy(src_hbm.at[pl.ds(base, CHUNK)], src_v)
    pltpu.sync_copy(dst_hbm.at[pl.ds(base, CHUNK)], dst_v)

    ones16 = jnp.ones((16,), jnp.float32)
    for i in range(8):
        ones_v[pl.ds(i * 16, 16)] = ones16
    zeros16 = jnp.zeros((16,), jnp.float32)
    for i in range(ZSL // 16):
        zero_v[pl.ds(i * 16, 16)] = zeros16
    for i in range(VECS):
        idx16 = dst_v[pl.ds(i * 16, 16)] * N + src_v[pl.ds(i * 16, 16)]
        idx_v[i // 8, pl.ds((i % 8) * 16, 16)] = idx16

    pltpu.sync_copy(zero_v, acc_sh.at[pl.ds(sid * ZSL, ZSL)])
    plsc.subcore_barrier()

    for j in range(16):
        pltpu.sync_copy(ones_v, acc_sh.at[idx_v.at[j]], add=True)
    plsc.subcore_barrier()

    pltpu.sync_copy(acc_sh.at[pl.ds(sid * ZSL, ZSL)],
                    out_hbm.at[cid, pl.ds(sid * ZSL, ZSL)])


@functools.lru_cache(maxsize=1)
def _edge_counts():
    mesh = plsc.VectorSubcoreMesh(core_axis_name="c", subcore_axis_name="s",
                                  num_cores=NC, num_subcores=NS)
    return pl.kernel(
        _edge_counts_body,
        out_type=jax.ShapeDtypeStruct((NC, ACC), jnp.float32),
        mesh=mesh,
        scratch_types=[
            pltpu.VMEM((CHUNK,), jnp.int32),
            pltpu.VMEM((CHUNK,), jnp.int32),
            pltpu.VMEM((16, 128), jnp.int32),
            pltpu.VMEM((128,), jnp.float32),
            pltpu.VMEM((ZSL,), jnp.float32),
            pltpu.VMEM_SHARED((ACC,), jnp.float32),
        ],
    )


_HI = lax.Precision.HIGHEST


def _dense_body(cp_ref, xp_ref, w1rel_ref, wrr_ref, brr_ref, w2rel_ref,
                w2root_ref, b2_ref, w3_ref, asrc_ref, adst_ref, b3_ref,
                h3_ref):
    cpv = cp_ref[...]
    c = cpv[0, :N, :] + cpv[1, :N, :]
    rows = lax.broadcasted_iota(jnp.int32, (N, N), 0)
    cols = lax.broadcasted_iota(jnp.int32, (N, N), 1)
    cp1 = c + (rows == cols).astype(jnp.float32)

    def mm(a, b, precision=_HI):
        return jnp.dot(a, b, precision=precision,
                       preferred_element_type=jnp.float32)

    xp = xp_ref[...]
    h1 = (mm(mm(c, xp), w1rel_ref[...], None) + mm(xp, wrr_ref[...], None)
          + brr_ref[...])
    h1 = jnp.maximum(h1, 0.0)
    h2 = (mm(mm(c, h1), w2rel_ref[...], None) + mm(h1, w2root_ref[...], None)
          + b2_ref[...])
    h2 = jnp.maximum(h2, 0.0)
    xp3 = mm(h2, w3_ref[...], None)
    mask = cp1 > 0.0
    acc = None
    for h in range(HEADS):
        xh = xp3[:, h * HID:(h + 1) * HID]
        a_s = asrc_ref[h:h + 1, :]
        a_d = adst_ref[h:h + 1, :]
        asr_row = lax.dot_general(a_s, xh, (((1,), (1,)), ((), ())),
                                  precision=_HI,
                                  preferred_element_type=jnp.float32)
        adt_col = lax.dot_general(xh, a_d, (((1,), (1,)), ((), ())),
                                  precision=_HI,
                                  preferred_element_type=jnp.float32)
        eh = adt_col + asr_row
        eh = jnp.where(eh >= 0.0, eh, 0.2 * eh)
        mh = jnp.max(jnp.where(mask, eh, -1e30), axis=1, keepdims=True)
        wh = cp1 * jnp.exp(jnp.minimum(eh - mh, 0.0))
        den = jnp.sum(wh, axis=1, keepdims=True)
        oh = mm(wh, xh) / den
        acc = oh if acc is None else acc + oh
    h3_ref[...] = jnp.maximum(acc * (1.0 / HEADS) + b3_ref[...], 0.0)


_dense = pl.pallas_call(
    _dense_body,
    out_shape=jax.ShapeDtypeStruct((N, HID), jnp.float32),
)


_KB = 2048
_NKB = (N * HID) // _KB


def _fc_body(f_ref, w_ref, b_ref, o_ref):
    @pl.when(pl.program_id(0) == 0)
    def _init():
        o_ref[...] = b_ref[...]

    o_ref[...] += jnp.dot(f_ref[...], w_ref[...],
                          preferred_element_type=jnp.float32)


_fc = pl.pallas_call(
    _fc_body,
    grid=(_NKB,),
    in_specs=[
        pl.BlockSpec((1, _KB), lambda k: (0, k)),
        pl.BlockSpec((_KB, OUT), lambda k: (k, 0)),
        pl.BlockSpec((1, OUT), lambda k: (0, 0)),
    ],
    out_specs=pl.BlockSpec((1, OUT), lambda k: (0, 0)),
    out_shape=jax.ShapeDtypeStruct((1, OUT), jnp.float32),
    compiler_params=pltpu.CompilerParams(
        dimension_semantics=("arbitrary",)),
)


def kernel(x, edge_index, W1_rel, W1_root, b1, W2_rel, W2_root, b2, W3,
           a_src, a_dst, b3, Wres, bres, Wfc, bfc):
    pad = EPAD - E
    src = jnp.concatenate([edge_index[0], jnp.zeros((pad,), jnp.int32)])
    dst = jnp.concatenate([edge_index[1], jnp.full((pad,), N, jnp.int32)])
    cpart = _edge_counts()(src, dst).reshape(NC, ROWS, N)

    xp = jnp.pad(x, ((0, 0), (0, HID - IN)))
    w1rel = jnp.pad(W1_rel, ((0, HID - IN), (0, 0)))
    wrr = jnp.pad(W1_root + Wres, ((0, HID - IN), (0, 0)))
    brr = (b1 + bres)[None, :]

    h3 = _dense(cpart, xp, w1rel, wrr, brr, W2_rel, W2_root, b2[None, :], W3,
                a_src, a_dst, b3[None, :])
    return _fc(h3.reshape(1, N * HID), Wfc, bfc[None, :])
